# baseline (speedup 1.0000x reference)
import jax
import jax.numpy as jnp
from jax.experimental import pallas as pl


_BB = 32


def _add_body(x_ref, t_ref, o_ref):
    o_ref[...] = x_ref[...] + t_ref[...][None]


def kernel(x, pos_table):
    B, L, D = x.shape
    R = (L * D) // 128
    xf = x.reshape(B, R, 128)
    tf = pos_table.reshape(R, 128)
    out = pl.pallas_call(
        _add_body,
        grid=(B // _BB,),
        in_specs=[
            pl.BlockSpec((_BB, R, 128), lambda i: (i, 0, 0)),
            pl.BlockSpec((R, 128), lambda i: (0, 0)),
        ],
        out_specs=pl.BlockSpec((_BB, R, 128), lambda i: (i, 0, 0)),
        out_shape=jax.ShapeDtypeStruct((B, R, 128), x.dtype),
    )(xf, tf)
    return out.reshape(B, L, D)
